# baseline (device time: 199262 ns/iter reference)
import jax
import jax.numpy as jnp
from jax import lax
from jax.experimental import pallas as pl
from jax.experimental.pallas import tpu as pltpu


def kernel(x, pi):
    def body(pi_ref, x_ref, out_ref, send_sem, recv_sem):
        my_i = lax.axis_index("i")
        dst = pi_ref[my_i]

        rdma = pltpu.make_async_remote_copy(
            src_ref=x_ref,
            dst_ref=out_ref,
            send_sem=send_sem,
            recv_sem=recv_sem,
            device_id=(dst,),
            device_id_type=pl.DeviceIdType.MESH,
        )
        rdma.start()
        rdma.wait()

    return pl.pallas_call(
        body,
        out_shape=jax.ShapeDtypeStruct(x.shape, jnp.float32),
        in_specs=[
            pl.BlockSpec(memory_space=pltpu.SMEM),
            pl.BlockSpec(memory_space=pltpu.VMEM),
        ],
        out_specs=pl.BlockSpec(memory_space=pltpu.VMEM),
        scratch_shapes=[
            pltpu.SemaphoreType.DMA,
            pltpu.SemaphoreType.DMA,
        ],
    )(pi, x)


# device time: 160918 ns/iter; 1.2383x vs baseline; 1.2383x over previous
import numpy as np

import jax
import jax.numpy as jnp
from jax import lax
from jax.experimental import pallas as pl
from jax.experimental.pallas import tpu as pltpu

N_DEV = 32


def _mid_table() -> np.ndarray:
    coords = [(x, y, z) for x in range(2) for y in range(4) for z in range(4)]
    ring = []
    for z in range(4):
        plane = sorted(c for c in coords if c[2] == z)
        for yi, y in enumerate(sorted({c[1] for c in plane})):
            ring.extend(sorted((c for c in plane if c[1] == y), reverse=bool(yi % 2)))
    cpos = {c: i for i, c in enumerate(ring)}
    tbl = np.zeros((N_DEV, N_DEV), dtype=np.int32)
    for s in range(N_DEV):
        for d in range(N_DEV):
            sx, sy, sz = ring[s]
            dx, dy, dz = ring[d]
            tbl[s, d] = cpos[(dx, dy, sz)]
    return tbl


def kernel(x, pi):
    mid_t = jnp.asarray(_mid_table())
    my_i = lax.axis_index("i").astype(jnp.int32)

    pi = pi.astype(jnp.int32)
    dst = pi[my_i]
    my_mid = mid_t[my_i, dst]
    mids_all = mid_t[jnp.arange(N_DEV), pi]
    stage_src = jnp.argmax(mids_all == my_i).astype(jnp.int32)
    stage_dst = pi[stage_src]
    params = jnp.stack([my_mid, stage_dst])

    def body(params_ref, x_ref, out_ref, stage_ref, s1, r1, s2, r2):
        my = lax.axis_index("i").astype(jnp.int32)
        mid = params_ref[0]
        fwd = params_ref[1]
        remote1 = mid != my
        remote2 = fwd != my

        @pl.when(remote1)
        def _():
            rdma1 = pltpu.make_async_remote_copy(
                src_ref=x_ref,
                dst_ref=stage_ref,
                send_sem=s1,
                recv_sem=r1,
                device_id=(mid,),
                device_id_type=pl.DeviceIdType.MESH,
            )
            rdma1.start()
            rdma1.wait()

        @pl.when(jnp.logical_not(remote1))
        def _():
            stage_ref[...] = x_ref[...]

        @pl.when(remote2)
        def _():
            rdma2 = pltpu.make_async_remote_copy(
                src_ref=stage_ref,
                dst_ref=out_ref,
                send_sem=s2,
                recv_sem=r2,
                device_id=(fwd,),
                device_id_type=pl.DeviceIdType.MESH,
            )
            rdma2.start()
            rdma2.wait()

        @pl.when(jnp.logical_not(remote2))
        def _():
            out_ref[...] = stage_ref[...]

    return pl.pallas_call(
        body,
        out_shape=jax.ShapeDtypeStruct(x.shape, jnp.float32),
        in_specs=[
            pl.BlockSpec(memory_space=pltpu.SMEM),
            pl.BlockSpec(memory_space=pltpu.VMEM),
        ],
        out_specs=pl.BlockSpec(memory_space=pltpu.VMEM),
        scratch_shapes=[
            pltpu.VMEM(x.shape, jnp.float32),
            pltpu.SemaphoreType.DMA,
            pltpu.SemaphoreType.DMA,
            pltpu.SemaphoreType.DMA,
            pltpu.SemaphoreType.DMA,
        ],
    )(params, x)


# device time: 122126 ns/iter; 1.6316x vs baseline; 1.3176x over previous
import numpy as np

import jax
import jax.numpy as jnp
from jax import lax
from jax.experimental import pallas as pl
from jax.experimental.pallas import tpu as pltpu

N_DEV = 32
CHUNKS = 8


def _mid_table() -> np.ndarray:
    coords = [(x, y, z) for x in range(2) for y in range(4) for z in range(4)]
    ring = []
    for z in range(4):
        plane = sorted(c for c in coords if c[2] == z)
        for yi, y in enumerate(sorted({c[1] for c in plane})):
            ring.extend(sorted((c for c in plane if c[1] == y), reverse=bool(yi % 2)))
    cpos = {c: i for i, c in enumerate(ring)}
    tbl = np.zeros((N_DEV, N_DEV), dtype=np.int32)
    for s in range(N_DEV):
        for d in range(N_DEV):
            dx, dy, _ = ring[d]
            tbl[s, d] = cpos[(dx, dy, ring[s][2])]
    return tbl


def kernel(x, pi):
    mid_t = jnp.asarray(_mid_table())
    my_i = lax.axis_index("i").astype(jnp.int32)

    pi = pi.astype(jnp.int32)
    dst = pi[my_i]
    my_mid = mid_t[my_i, dst]
    mids_all = mid_t[jnp.arange(N_DEV), pi]
    stage_src = jnp.argmax(mids_all == my_i).astype(jnp.int32)
    stage_dst = pi[stage_src]
    params = jnp.stack([my_mid, stage_dst])

    rows = x.shape[1] // CHUNKS

    def body(params_ref, x_ref, out_ref, stage_ref, s1, r1, s2, r2):
        my = lax.axis_index("i").astype(jnp.int32)
        mid = params_ref[0]
        fwd = params_ref[1]
        remote1 = mid != my
        remote2 = fwd != my

        def chunk(ref, c):
            return ref.at[:, pl.ds(c * rows, rows), :]

        def rdma1(c):
            return pltpu.make_async_remote_copy(
                src_ref=chunk(x_ref, c),
                dst_ref=chunk(stage_ref, c),
                send_sem=s1.at[c],
                recv_sem=r1.at[c],
                device_id=(mid,),
                device_id_type=pl.DeviceIdType.MESH,
            )

        def rdma2(c):
            return pltpu.make_async_remote_copy(
                src_ref=chunk(stage_ref, c),
                dst_ref=chunk(out_ref, c),
                send_sem=s2.at[c],
                recv_sem=r2.at[c],
                device_id=(fwd,),
                device_id_type=pl.DeviceIdType.MESH,
            )

        @pl.when(remote1)
        def _():
            for c in range(CHUNKS):
                rdma1(c).start()

        @pl.when(jnp.logical_not(remote1))
        def _():
            stage_ref[...] = x_ref[...]

        for c in range(CHUNKS):
            @pl.when(remote1)
            def _(c=c):
                rdma1(c).wait_recv()

            @pl.when(remote2)
            def _(c=c):
                rdma2(c).start()

            @pl.when(jnp.logical_not(remote2))
            def _(c=c):
                chunk(out_ref, c)[...] = chunk(stage_ref, c)[...]

        @pl.when(remote1)
        def _():
            for c in range(CHUNKS):
                rdma1(c).wait_send()

        @pl.when(remote2)
        def _():
            for c in range(CHUNKS):
                rdma2(c).wait()

    return pl.pallas_call(
        body,
        out_shape=jax.ShapeDtypeStruct(x.shape, jnp.float32),
        in_specs=[
            pl.BlockSpec(memory_space=pltpu.SMEM),
            pl.BlockSpec(memory_space=pltpu.VMEM),
        ],
        out_specs=pl.BlockSpec(memory_space=pltpu.VMEM),
        scratch_shapes=[
            pltpu.VMEM(x.shape, jnp.float32),
            pltpu.SemaphoreType.DMA((CHUNKS,)),
            pltpu.SemaphoreType.DMA((CHUNKS,)),
            pltpu.SemaphoreType.DMA((CHUNKS,)),
            pltpu.SemaphoreType.DMA((CHUNKS,)),
        ],
    )(params, x)


# device time: 116020 ns/iter; 1.7175x vs baseline; 1.0526x over previous
import numpy as np

import jax
import jax.numpy as jnp
from jax import lax
from jax.experimental import pallas as pl
from jax.experimental.pallas import tpu as pltpu

N_DEV = 32
CHUNKS = 16


def _mid_table() -> np.ndarray:
    coords = [(x, y, z) for x in range(2) for y in range(4) for z in range(4)]
    ring = []
    for z in range(4):
        plane = sorted(c for c in coords if c[2] == z)
        for yi, y in enumerate(sorted({c[1] for c in plane})):
            ring.extend(sorted((c for c in plane if c[1] == y), reverse=bool(yi % 2)))
    cpos = {c: i for i, c in enumerate(ring)}
    tbl = np.zeros((N_DEV, N_DEV), dtype=np.int32)
    for s in range(N_DEV):
        for d in range(N_DEV):
            dx, dy, _ = ring[d]
            tbl[s, d] = cpos[(dx, dy, ring[s][2])]
    return tbl


def kernel(x, pi):
    mid_t = jnp.asarray(_mid_table())
    my_i = lax.axis_index("i").astype(jnp.int32)

    pi = pi.astype(jnp.int32)
    dst = pi[my_i]
    my_mid = mid_t[my_i, dst]
    mids_all = mid_t[jnp.arange(N_DEV), pi]
    stage_src = jnp.argmax(mids_all == my_i).astype(jnp.int32)
    stage_dst = pi[stage_src]
    src = jnp.argsort(pi)[my_i].astype(jnp.int32)
    m_star = mid_t[src, my_i]
    params = jnp.stack([my_mid, stage_dst, stage_src, m_star])

    rows = x.shape[1] // CHUNKS

    def body(params_ref, x_ref, out_ref, stage_ref, s1, r1, s2, r2):
        my = lax.axis_index("i").astype(jnp.int32)
        mid = params_ref[0]
        fwd = params_ref[1]
        remote1 = mid != my
        remote2 = fwd != my

        barrier_sem = pltpu.get_barrier_semaphore()
        for p in range(4):
            pl.semaphore_signal(
                barrier_sem, inc=1,
                device_id=(params_ref[p],),
                device_id_type=pl.DeviceIdType.MESH,
            )
        pl.semaphore_wait(barrier_sem, 4)

        def chunk(ref, c):
            return ref.at[:, pl.ds(c * rows, rows), :]

        def rdma1(c):
            return pltpu.make_async_remote_copy(
                src_ref=chunk(x_ref, c),
                dst_ref=chunk(stage_ref, c),
                send_sem=s1.at[c],
                recv_sem=r1.at[c],
                device_id=(mid,),
                device_id_type=pl.DeviceIdType.MESH,
            )

        def rdma2(c):
            return pltpu.make_async_remote_copy(
                src_ref=chunk(stage_ref, c),
                dst_ref=chunk(out_ref, c),
                send_sem=s2.at[c],
                recv_sem=r2.at[c],
                device_id=(fwd,),
                device_id_type=pl.DeviceIdType.MESH,
            )

        @pl.when(remote1)
        def _():
            for c in range(CHUNKS):
                rdma1(c).start()

        @pl.when(jnp.logical_not(remote1))
        def _():
            stage_ref[...] = x_ref[...]

        for c in range(CHUNKS):
            @pl.when(remote1)
            def _(c=c):
                rdma1(c).wait_recv()

            @pl.when(remote2)
            def _(c=c):
                rdma2(c).start()

            @pl.when(jnp.logical_not(remote2))
            def _(c=c):
                chunk(out_ref, c)[...] = chunk(stage_ref, c)[...]

        @pl.when(remote1)
        def _():
            for c in range(CHUNKS):
                rdma1(c).wait_send()

        @pl.when(remote2)
        def _():
            for c in range(CHUNKS):
                rdma2(c).wait()

    return pl.pallas_call(
        body,
        out_shape=jax.ShapeDtypeStruct(x.shape, jnp.float32),
        in_specs=[
            pl.BlockSpec(memory_space=pltpu.SMEM),
            pl.BlockSpec(memory_space=pltpu.VMEM),
        ],
        out_specs=pl.BlockSpec(memory_space=pltpu.VMEM),
        scratch_shapes=[
            pltpu.VMEM(x.shape, jnp.float32),
            pltpu.SemaphoreType.DMA((CHUNKS,)),
            pltpu.SemaphoreType.DMA((CHUNKS,)),
            pltpu.SemaphoreType.DMA((CHUNKS,)),
            pltpu.SemaphoreType.DMA((CHUNKS,)),
        ],
        compiler_params=pltpu.CompilerParams(collective_id=0),
    )(params, x)
